# baseline (speedup 1.0000x reference)
import functools

import jax
import jax.numpy as jnp
from jax import lax
from jax.experimental import pallas as pl
from jax.experimental.pallas import tpu as pltpu
from jax.experimental.pallas import tpu_sc as plsc

N = 10000
E = 320000
D = 128
R = 7
DH = 64

NC = 2
NS = 16
NW = NC * NS
C = 48
EP = 322560
EPW = EP // NW
NCHUNK = EPW // C
NPAD = 10240
ROWS_PER_TILE = NPAD // NS
ZROWS = 64

BN = 1000


def _pre_body(h_ref, wq_ref, wk_ref, wqp_ref, wrelp_ref, ws_ref, b_ref,
              qk_ref, hwq_ref, hsb_ref):
    h = h_ref[...]
    wqk = jnp.concatenate([wq_ref[...], wk_ref[...]], axis=1)
    qk_ref[...] = jnp.dot(h, wqk, preferred_element_type=jnp.float32)
    hsb_ref[...] = (
        jnp.dot(h, ws_ref[...], preferred_element_type=jnp.float32) + b_ref[...]
    )
    q_bf = jnp.dot(
        h, wqp_ref[...], preferred_element_type=jnp.float32
    ).astype(jnp.bfloat16)
    zpad = jnp.zeros((q_bf.shape[0], DH), jnp.bfloat16)
    for r in range(R):
        hw_bf = jnp.dot(
            h, wrelp_ref[r], preferred_element_type=jnp.float32
        ).astype(jnp.bfloat16)
        hwq_ref[r] = jnp.concatenate([hw_bf, q_bf, zpad], axis=1)


@jax.jit
def _tc_pre(h, wq, wk, wqp, wrelp, ws, b2):
    return pl.pallas_call(
        _pre_body,
        grid=(N // BN,),
        in_specs=[
            pl.BlockSpec((BN, D), lambda i: (i, 0)),
            pl.BlockSpec((D, DH), lambda i: (0, 0)),
            pl.BlockSpec((D, DH), lambda i: (0, 0)),
            pl.BlockSpec((D, DH), lambda i: (0, 0)),
            pl.BlockSpec((R, D, D), lambda i: (0, 0, 0)),
            pl.BlockSpec((D, D), lambda i: (0, 0)),
            pl.BlockSpec((1, D), lambda i: (0, 0)),
        ],
        out_specs=[
            pl.BlockSpec((BN, 2 * DH), lambda i: (i, 0)),
            pl.BlockSpec((R, BN, 2 * D), lambda i: (0, i, 0)),
            pl.BlockSpec((BN, D), lambda i: (i, 0)),
        ],
        out_shape=[
            jax.ShapeDtypeStruct((N, 2 * DH), jnp.float32),
            jax.ShapeDtypeStruct((R, N, 2 * D), jnp.bfloat16),
            jax.ShapeDtypeStruct((N, D), jnp.float32),
        ],
    )(h, wq, wk, wqp, wrelp, ws, b2)


def _combine_body(acc_ref, hsb_ref, h_ref, gf_ref):
    hv = jnp.maximum(acc_ref[0] + acc_ref[1] + hsb_ref[...], 0.0)
    h_ref[...] = hv

    @pl.when(pl.program_id(0) == 0)
    def _():
        gf_ref[...] = jnp.zeros_like(gf_ref)

    gf_ref[...] += jnp.sum(hv, axis=0, keepdims=True)


@jax.jit
def _tc_combine(accp, hsb):
    return pl.pallas_call(
        _combine_body,
        grid=(N // BN,),
        in_specs=[
            pl.BlockSpec((2, BN, D), lambda i: (0, i, 0)),
            pl.BlockSpec((BN, D), lambda i: (i, 0)),
        ],
        out_specs=[
            pl.BlockSpec((BN, D), lambda i: (i, 0)),
            pl.BlockSpec((1, D), lambda i: (0, 0)),
        ],
        out_shape=[
            jax.ShapeDtypeStruct((N, D), jnp.float32),
            jax.ShapeDtypeStruct((1, D), jnp.float32),
        ],
    )(accp, hsb)


NGROUP = C // 16


def _edge_body(qk_hbm, hw_hbm, src_hbm, dst_hbm, et_hbm, out_hbm,
               sbufs, dbufs, gbufs, dscats, kbufs, mbufs, obufs, zbuf, acc,
               semi, semg, sems):
    cid = lax.axis_index("c")
    sid = lax.axis_index("s")
    wid = sid * NC + cid

    zv = jnp.zeros((16,), jnp.float32)

    def _zero_row(r, _):
        for j in range(D // 16):
            zbuf[r, pl.ds(j * 16, 16)] = zv
        return 0

    lax.fori_loop(0, ZROWS, _zero_row, 0)
    for p in range(ROWS_PER_TILE // ZROWS):
        pltpu.sync_copy(zbuf, acc.at[pl.ds(sid * ROWS_PER_TILE + p * ZROWS, ZROWS)])
    plsc.subcore_barrier()

    iota16 = lax.iota(jnp.int32, 16)

    def issue_idx(s, ch):
        base = wid * EPW + ch * C
        pltpu.async_copy(src_hbm.at[pl.ds(base, C)], sbufs[s], semi[s])
        pltpu.async_copy(dst_hbm.at[pl.ds(base, C)], dbufs[s], semi[s])
        pltpu.async_copy(et_hbm.at[pl.ds(base, C)], gbufs[s], semi[s])

    def wait_idx(s):
        pltpu.make_async_copy(src_hbm.at[pl.ds(0, C)], sbufs[s], semi[s]).wait()
        pltpu.make_async_copy(dst_hbm.at[pl.ds(0, C)], dbufs[s], semi[s]).wait()
        pltpu.make_async_copy(et_hbm.at[pl.ds(0, C)], gbufs[s], semi[s]).wait()

    def issue_gathers(s):
        for g in range(NGROUP):
            ev = gbufs[s][pl.ds(g * 16, 16)]
            sv = sbufs[s][pl.ds(g * 16, 16)]
            gbufs[s][pl.ds(g * 16, 16)] = ev * N + sv
        pltpu.async_copy(qk_hbm.at[dbufs[s]], kbufs[s], semg[s])
        pltpu.async_copy(hw_hbm.at[gbufs[s]], mbufs[s], semg[s])

    def wait_gathers(s):
        pltpu.make_async_copy(qk_hbm.at[dbufs[s]], kbufs[s], semg[s]).wait()
        pltpu.make_async_copy(hw_hbm.at[gbufs[s]], mbufs[s], semg[s]).wait()

    def compute(s):
        kbuf, mbuf, obuf = kbufs[s], mbufs[s], obufs[s]

        def _group(g, _):
            def _dot(i, zv):
                e = g * 16 + i
                p = jnp.zeros((16,), jnp.float32)
                for j in range(DH // 32):
                    qa, qb = plsc.unpack(
                        plsc.bitcast(mbuf[e, pl.ds(DH + j * 16, 16)],
                                     jnp.bfloat16),
                        format=plsc.PackFormat.INTERLEAVED)
                    p = (p + qa * kbuf[e, pl.ds(DH + j * 32, 16)]
                         + qb * kbuf[e, pl.ds(DH + j * 32 + 16, 16)])
                return jnp.where(iota16 == i, jnp.sum(p), zv)

            zv = lax.fori_loop(0, 16, _dot, jnp.zeros((16,), jnp.float32))
            zv = zv * 0.125
            w = 1.0 / (1.0 + jnp.exp(-zv))

            def _scale(i, _):
                e = g * 16 + i
                wi = jnp.sum(jnp.where(iota16 == i, w, 0.0))
                for j in range(D // 32):
                    ha, hb = plsc.unpack(
                        plsc.bitcast(mbuf[e, pl.ds(j * 16, 16)],
                                     jnp.bfloat16),
                        format=plsc.PackFormat.INTERLEAVED)
                    obuf[e, pl.ds(j * 32, 16)] = ha * wi
                    obuf[e, pl.ds(j * 32 + 16, 16)] = hb * wi
                return 0

            lax.fori_loop(0, 16, _scale, 0)
            return 0

        lax.fori_loop(0, NGROUP, _group, 0)

    def snap_dst(s):
        for g in range(NGROUP):
            dscats[s][pl.ds(g * 16, 16)] = dbufs[s][pl.ds(g * 16, 16)]

    def issue_scatter(s):
        pltpu.async_copy(obufs[s], acc.at[dscats[s]], sems[s], add=True)

    def wait_scatter(s):
        pltpu.make_async_copy(obufs[s], acc.at[dscats[s]], sems[s]).wait()

    NP_ = NCHUNK // 2
    issue_idx(0, 0)
    issue_idx(1, 1)
    wait_idx(0)
    issue_gathers(0)

    def _pair(p, _):
        wait_gathers(0)
        snap_dst(0)

        @pl.when(p > 0)
        def _():
            wait_scatter(1)
        wait_idx(1)
        issue_gathers(1)

        @pl.when(p < NP_ - 1)
        def _():
            issue_idx(0, 2 * p + 2)
        compute(0)
        issue_scatter(0)

        wait_gathers(1)
        snap_dst(1)
        wait_scatter(0)

        @pl.when(p < NP_ - 1)
        def _():
            wait_idx(0)
            issue_gathers(0)
            issue_idx(1, 2 * p + 3)
        compute(1)
        issue_scatter(1)
        return 0

    lax.fori_loop(0, NP_, _pair, 0)
    wait_scatter(1)

    plsc.subcore_barrier()
    pltpu.sync_copy(
        acc.at[pl.ds(sid * ROWS_PER_TILE, ROWS_PER_TILE)],
        out_hbm.at[cid, pl.ds(sid * ROWS_PER_TILE, ROWS_PER_TILE)],
    )


@jax.jit
def _sc_edge(qk, hwq_flat, src, dst, et):
    mesh = plsc.VectorSubcoreMesh(core_axis_name="c", subcore_axis_name="s")
    idx_t = pltpu.VMEM((C,), jnp.int32)
    k_t = pltpu.VMEM((C, D), jnp.float32)
    m_t = pltpu.VMEM((C, D), jnp.int32)
    o_t = pltpu.VMEM((C, D), jnp.float32)
    f = functools.partial(
        pl.kernel,
        mesh=mesh,
        compiler_params=pltpu.CompilerParams(needs_layout_passes=False),
        out_type=jax.ShapeDtypeStruct((2, NPAD, D), jnp.float32),
        scratch_types=[
            (idx_t, idx_t),
            (idx_t, idx_t),
            (idx_t, idx_t),
            (idx_t, idx_t),
            (k_t, k_t),
            (m_t, m_t),
            (o_t, o_t),
            pltpu.VMEM((ZROWS, D), jnp.float32),
            pltpu.VMEM_SHARED((NPAD, D), jnp.float32),
            (pltpu.SemaphoreType.DMA, pltpu.SemaphoreType.DMA),
            (pltpu.SemaphoreType.DMA, pltpu.SemaphoreType.DMA),
            (pltpu.SemaphoreType.DMA, pltpu.SemaphoreType.DMA),
        ],
    )(_edge_body)
    return f(qk, hwq_flat, src, dst, et)


def kernel(x, Wq0, Wk0, Wrel0, b0, Ws0, Wq1, Wk1, Wrel1, b1, Ws1,
           Wq2, Wk2, Wrel2, b2, Ws2, edge_index, edge_type):
    npad_e = EP - E
    src = jnp.concatenate(
        [edge_index[0].astype(jnp.int32), jnp.zeros((npad_e,), jnp.int32)])
    dst = jnp.concatenate(
        [edge_index[1].astype(jnp.int32),
         jnp.full((npad_e,), NPAD - 1, jnp.int32)])
    et = jnp.concatenate(
        [edge_type.astype(jnp.int32), jnp.zeros((npad_e,), jnp.int32)])

    def _block_perm(width):
        perm = [0] * width
        for blk in range(width // 32):
            for i in range(16):
                perm[blk * 32 + 2 * i] = blk * 32 + i
                perm[blk * 32 + 2 * i + 1] = blk * 32 + 16 + i
        return jnp.asarray(perm, jnp.int32)

    p64 = _block_perm(DH)
    p128 = _block_perm(D)

    wqs = jnp.stack([Wq0, Wq1, Wq2])
    wks = jnp.stack([Wk0, Wk1, Wk2])
    wqps = wqs[:, :, p64]
    wrels = jnp.stack([Wrel0.reshape(R, D, D), Wrel1.reshape(R, D, D),
                       Wrel2.reshape(R, D, D)])
    wrelps = wrels[:, :, :, p128]
    bs = jnp.stack([b0.reshape(1, D), b1.reshape(1, D), b2.reshape(1, D)])
    wss = jnp.stack([Ws0, Ws1, Ws2])

    def _layer_step(carry, ws):
        h, _ = carry
        wq, wk, wqp, wrelp, b2, w_s = ws
        qk, hwq, hsb = _tc_pre(h, wq, wk, wqp, wrelp, w_s, b2)
        hwq_i32 = jax.lax.bitcast_convert_type(
            hwq.reshape(R * N, D, 2), jnp.int32)
        accp = _sc_edge(qk, hwq_i32, src, dst, et)
        h_new, gf = _tc_combine(accp, hsb)
        return (h_new, gf), None

    gf0 = jnp.zeros((1, D), jnp.float32)
    (h, gf), _ = lax.scan(_layer_step, (x, gf0),
                          (wqs, wks, wqps, wrelps, bs, wss))
    return gf, h


# speedup vs baseline: 1.0813x; 1.0813x over previous
import functools

import jax
import jax.numpy as jnp
from jax import lax
from jax.experimental import pallas as pl
from jax.experimental.pallas import tpu as pltpu
from jax.experimental.pallas import tpu_sc as plsc

N = 10000
E = 320000
D = 128
R = 7
DH = 64

NC = 2
NS = 16
NW = NC * NS
C = 48
EP = 322560
EPW = EP // NW
NCHUNK = EPW // C
NPAD = 10240
ROWS_PER_TILE = NPAD // NS
ZROWS = 64

BN = 1000


def _pre_body(h_ref, wq_ref, wk_ref, wqp_ref, wrelp_ref, ws_ref, b_ref,
              qk_ref, hwq_ref, hsb_ref):
    h = h_ref[...]
    wqk = jnp.concatenate([wq_ref[...], wk_ref[...]], axis=1)
    qk_ref[...] = jnp.dot(h, wqk, preferred_element_type=jnp.float32)
    hsb_ref[...] = (
        jnp.dot(h, ws_ref[...], preferred_element_type=jnp.float32) + b_ref[...]
    )
    q_bf = jnp.dot(
        h, wqp_ref[...], preferred_element_type=jnp.float32
    ).astype(jnp.bfloat16)
    zpad = jnp.zeros((q_bf.shape[0], DH), jnp.bfloat16)
    for r in range(R):
        hw_bf = jnp.dot(
            h, wrelp_ref[r], preferred_element_type=jnp.float32
        ).astype(jnp.bfloat16)
        hwq_ref[r] = jnp.concatenate([hw_bf, q_bf, zpad], axis=1)


@jax.jit
def _tc_pre(h, wq, wk, wqp, wrelp, ws, b2):
    return pl.pallas_call(
        _pre_body,
        grid=(N // BN,),
        in_specs=[
            pl.BlockSpec((BN, D), lambda i: (i, 0)),
            pl.BlockSpec((D, DH), lambda i: (0, 0)),
            pl.BlockSpec((D, DH), lambda i: (0, 0)),
            pl.BlockSpec((D, DH), lambda i: (0, 0)),
            pl.BlockSpec((R, D, D), lambda i: (0, 0, 0)),
            pl.BlockSpec((D, D), lambda i: (0, 0)),
            pl.BlockSpec((1, D), lambda i: (0, 0)),
        ],
        out_specs=[
            pl.BlockSpec((BN, 2 * DH), lambda i: (i, 0)),
            pl.BlockSpec((R, BN, 2 * D), lambda i: (0, i, 0)),
            pl.BlockSpec((BN, D), lambda i: (i, 0)),
        ],
        out_shape=[
            jax.ShapeDtypeStruct((N, 2 * DH), jnp.float32),
            jax.ShapeDtypeStruct((R, N, 2 * D), jnp.bfloat16),
            jax.ShapeDtypeStruct((N, D), jnp.float32),
        ],
    )(h, wq, wk, wqp, wrelp, ws, b2)


def _combine_body(acc_ref, hsb_ref, h_ref, gf_ref):
    hv = jnp.maximum(acc_ref[0] + acc_ref[1] + hsb_ref[...], 0.0)
    h_ref[...] = hv

    @pl.when(pl.program_id(0) == 0)
    def _():
        gf_ref[...] = jnp.zeros_like(gf_ref)

    gf_ref[...] += jnp.sum(hv, axis=0, keepdims=True)


@jax.jit
def _tc_combine(accp, hsb):
    return pl.pallas_call(
        _combine_body,
        grid=(N // BN,),
        in_specs=[
            pl.BlockSpec((2, BN, D), lambda i: (0, i, 0)),
            pl.BlockSpec((BN, D), lambda i: (i, 0)),
        ],
        out_specs=[
            pl.BlockSpec((BN, D), lambda i: (i, 0)),
            pl.BlockSpec((1, D), lambda i: (0, 0)),
        ],
        out_shape=[
            jax.ShapeDtypeStruct((N, D), jnp.float32),
            jax.ShapeDtypeStruct((1, D), jnp.float32),
        ],
    )(accp, hsb)


NGROUP = C // 16


def _edge_body(qk_hbm, hw_hbm, src_hbm, dst_hbm, et_hbm, out_hbm,
               sbufs, dbufs, gbufs, dscats, kbufs, mbufs, obufs, zbuf, acc,
               semi, semg, sems):
    cid = lax.axis_index("c")
    sid = lax.axis_index("s")
    wid = sid * NC + cid

    zv = jnp.zeros((16,), jnp.float32)

    def _zero_row(r, _):
        for j in range(D // 16):
            zbuf[r, pl.ds(j * 16, 16)] = zv
        return 0

    lax.fori_loop(0, ZROWS, _zero_row, 0)
    for p in range(ROWS_PER_TILE // ZROWS):
        pltpu.sync_copy(zbuf, acc.at[pl.ds(sid * ROWS_PER_TILE + p * ZROWS, ZROWS)])
    plsc.subcore_barrier()

    iota16 = lax.iota(jnp.int32, 16)

    def issue_idx(s, ch):
        base = wid * EPW + ch * C
        pltpu.async_copy(src_hbm.at[pl.ds(base, C)], sbufs[s], semi[s])
        pltpu.async_copy(dst_hbm.at[pl.ds(base, C)], dbufs[s], semi[s])
        pltpu.async_copy(et_hbm.at[pl.ds(base, C)], gbufs[s], semi[s])

    def wait_idx(s):
        pltpu.make_async_copy(src_hbm.at[pl.ds(0, C)], sbufs[s], semi[s]).wait()
        pltpu.make_async_copy(dst_hbm.at[pl.ds(0, C)], dbufs[s], semi[s]).wait()
        pltpu.make_async_copy(et_hbm.at[pl.ds(0, C)], gbufs[s], semi[s]).wait()

    def issue_gathers(s):
        for g in range(NGROUP):
            ev = gbufs[s][pl.ds(g * 16, 16)]
            sv = sbufs[s][pl.ds(g * 16, 16)]
            gbufs[s][pl.ds(g * 16, 16)] = ev * N + sv
        pltpu.async_copy(qk_hbm.at[dbufs[s]], kbufs[s], semg[s])
        pltpu.async_copy(hw_hbm.at[gbufs[s]], mbufs[s], semg[s])

    def wait_gathers(s):
        pltpu.make_async_copy(qk_hbm.at[dbufs[s]], kbufs[s], semg[s]).wait()
        pltpu.make_async_copy(hw_hbm.at[gbufs[s]], mbufs[s], semg[s]).wait()

    def compute(s):
        kbuf, mbuf, obuf = kbufs[s], mbufs[s], obufs[s]

        def _group(g, _):
            def _dot4(ii, zv):
                for t in range(4):
                    i = ii * 4 + t
                    e = g * 16 + i
                    p = jnp.zeros((16,), jnp.float32)
                    for j in range(DH // 32):
                        qa, qb = plsc.unpack(
                            plsc.bitcast(mbuf[e, pl.ds(DH + j * 16, 16)],
                                         jnp.bfloat16),
                            format=plsc.PackFormat.INTERLEAVED)
                        p = (p + qa * kbuf[e, pl.ds(DH + j * 32, 16)]
                             + qb * kbuf[e, pl.ds(DH + j * 32 + 16, 16)])
                    zv = jnp.where(iota16 == i, jnp.sum(p), zv)
                return zv

            zv = lax.fori_loop(0, 4, _dot4, jnp.zeros((16,), jnp.float32))
            zv = zv * 0.125
            w = 1.0 / (1.0 + jnp.exp(-zv))
            for i in range(16):
                e = g * 16 + i
                wi = w[i]
                for j in range(D // 32):
                    ha, hb = plsc.unpack(
                        plsc.bitcast(mbuf[e, pl.ds(j * 16, 16)],
                                     jnp.bfloat16),
                        format=plsc.PackFormat.INTERLEAVED)
                    obuf[e, pl.ds(j * 32, 16)] = ha * wi
                    obuf[e, pl.ds(j * 32 + 16, 16)] = hb * wi
            return 0

        lax.fori_loop(0, NGROUP, _group, 0)

    def snap_dst(s):
        for g in range(NGROUP):
            dscats[s][pl.ds(g * 16, 16)] = dbufs[s][pl.ds(g * 16, 16)]

    def issue_scatter(s):
        pltpu.async_copy(obufs[s], acc.at[dscats[s]], sems[s], add=True)

    def wait_scatter(s):
        pltpu.make_async_copy(obufs[s], acc.at[dscats[s]], sems[s]).wait()

    NP_ = NCHUNK // 2
    issue_idx(0, 0)
    issue_idx(1, 1)
    wait_idx(0)
    issue_gathers(0)

    def _pair(p, _):
        wait_gathers(0)
        snap_dst(0)

        @pl.when(p > 0)
        def _():
            wait_scatter(1)
        wait_idx(1)
        issue_gathers(1)

        @pl.when(p < NP_ - 1)
        def _():
            issue_idx(0, 2 * p + 2)
        compute(0)
        issue_scatter(0)

        wait_gathers(1)
        snap_dst(1)
        wait_scatter(0)

        @pl.when(p < NP_ - 1)
        def _():
            wait_idx(0)
            issue_gathers(0)
            issue_idx(1, 2 * p + 3)
        compute(1)
        issue_scatter(1)
        return 0

    lax.fori_loop(0, NP_, _pair, 0)
    wait_scatter(1)

    plsc.subcore_barrier()
    pltpu.sync_copy(
        acc.at[pl.ds(sid * ROWS_PER_TILE, ROWS_PER_TILE)],
        out_hbm.at[cid, pl.ds(sid * ROWS_PER_TILE, ROWS_PER_TILE)],
    )


@jax.jit
def _sc_edge(qk, hwq_flat, src, dst, et):
    mesh = plsc.VectorSubcoreMesh(core_axis_name="c", subcore_axis_name="s")
    idx_t = pltpu.VMEM((C,), jnp.int32)
    k_t = pltpu.VMEM((C, D), jnp.float32)
    m_t = pltpu.VMEM((C, D), jnp.int32)
    o_t = pltpu.VMEM((C, D), jnp.float32)
    f = functools.partial(
        pl.kernel,
        mesh=mesh,
        compiler_params=pltpu.CompilerParams(needs_layout_passes=False),
        out_type=jax.ShapeDtypeStruct((2, NPAD, D), jnp.float32),
        scratch_types=[
            (idx_t, idx_t),
            (idx_t, idx_t),
            (idx_t, idx_t),
            (idx_t, idx_t),
            (k_t, k_t),
            (m_t, m_t),
            (o_t, o_t),
            pltpu.VMEM((ZROWS, D), jnp.float32),
            pltpu.VMEM_SHARED((NPAD, D), jnp.float32),
            (pltpu.SemaphoreType.DMA, pltpu.SemaphoreType.DMA),
            (pltpu.SemaphoreType.DMA, pltpu.SemaphoreType.DMA),
            (pltpu.SemaphoreType.DMA, pltpu.SemaphoreType.DMA),
        ],
    )(_edge_body)
    return f(qk, hwq_flat, src, dst, et)


def kernel(x, Wq0, Wk0, Wrel0, b0, Ws0, Wq1, Wk1, Wrel1, b1, Ws1,
           Wq2, Wk2, Wrel2, b2, Ws2, edge_index, edge_type):
    npad_e = EP - E
    src = jnp.concatenate(
        [edge_index[0].astype(jnp.int32), jnp.zeros((npad_e,), jnp.int32)])
    dst = jnp.concatenate(
        [edge_index[1].astype(jnp.int32),
         jnp.full((npad_e,), NPAD - 1, jnp.int32)])
    et = jnp.concatenate(
        [edge_type.astype(jnp.int32), jnp.zeros((npad_e,), jnp.int32)])

    def _block_perm(width):
        perm = [0] * width
        for blk in range(width // 32):
            for i in range(16):
                perm[blk * 32 + 2 * i] = blk * 32 + i
                perm[blk * 32 + 2 * i + 1] = blk * 32 + 16 + i
        return jnp.asarray(perm, jnp.int32)

    p64 = _block_perm(DH)
    p128 = _block_perm(D)

    wqs = jnp.stack([Wq0, Wq1, Wq2])
    wks = jnp.stack([Wk0, Wk1, Wk2])
    wqps = wqs[:, :, p64]
    wrels = jnp.stack([Wrel0.reshape(R, D, D), Wrel1.reshape(R, D, D),
                       Wrel2.reshape(R, D, D)])
    wrelps = wrels[:, :, :, p128]
    bs = jnp.stack([b0.reshape(1, D), b1.reshape(1, D), b2.reshape(1, D)])
    wss = jnp.stack([Ws0, Ws1, Ws2])

    def _layer_step(carry, ws):
        h, _ = carry
        wq, wk, wqp, wrelp, b2, w_s = ws
        qk, hwq, hsb = _tc_pre(h, wq, wk, wqp, wrelp, w_s, b2)
        hwq_i32 = jax.lax.bitcast_convert_type(
            hwq.reshape(R * N, D, 2), jnp.int32)
        accp = _sc_edge(qk, hwq_i32, src, dst, et)
        h_new, gf = _tc_combine(accp, hsb)
        return (h_new, gf), None

    gf0 = jnp.zeros((1, D), jnp.float32)
    (h, gf), _ = lax.scan(_layer_step, (x, gf0),
                          (wqs, wks, wqps, wrelps, bs, wss))
    return gf, h


# speedup vs baseline: 2.2338x; 2.0658x over previous
import functools

import jax
import jax.numpy as jnp
from jax import lax
from jax.experimental import pallas as pl
from jax.experimental.pallas import tpu as pltpu
from jax.experimental.pallas import tpu_sc as plsc

N = 10000
E = 320000
D = 128
R = 7
DH = 64

NC = 2
NS = 16
NW = NC * NS
C = 48
EP = 322560
EPW = EP // NW
NCHUNK = EPW // C
NPAD = 10240
ROWS_PER_TILE = NPAD // NS
ZROWS = 64

BN = 1000


def _pre_body(h_ref, wq_ref, wk_ref, wrel_ref, ws_ref, b_ref,
              qk_ref, hw_ref, hsb_ref):
    h = h_ref[...]
    wqk = jnp.concatenate([wq_ref[...], wk_ref[...]], axis=1)
    qk_ref[...] = jnp.dot(h, wqk, preferred_element_type=jnp.float32)
    hsb_ref[...] = (
        jnp.dot(h, ws_ref[...], preferred_element_type=jnp.float32) + b_ref[...]
    )
    for r in range(R):
        hw_ref[r] = jnp.dot(h, wrel_ref[r], preferred_element_type=jnp.float32)


@jax.jit
def _tc_pre(h, wq, wk, wrel, ws, b2):
    return pl.pallas_call(
        _pre_body,
        grid=(N // BN,),
        in_specs=[
            pl.BlockSpec((BN, D), lambda i: (i, 0)),
            pl.BlockSpec((D, DH), lambda i: (0, 0)),
            pl.BlockSpec((D, DH), lambda i: (0, 0)),
            pl.BlockSpec((R, D, D), lambda i: (0, 0, 0)),
            pl.BlockSpec((D, D), lambda i: (0, 0)),
            pl.BlockSpec((1, D), lambda i: (0, 0)),
        ],
        out_specs=[
            pl.BlockSpec((BN, 2 * DH), lambda i: (i, 0)),
            pl.BlockSpec((R, BN, D), lambda i: (0, i, 0)),
            pl.BlockSpec((BN, D), lambda i: (i, 0)),
        ],
        out_shape=[
            jax.ShapeDtypeStruct((N, 2 * DH), jnp.float32),
            jax.ShapeDtypeStruct((R, N, D), jnp.float32),
            jax.ShapeDtypeStruct((N, D), jnp.float32),
        ],
    )(h, wq, wk, wrel, ws, b2)


def _combine_body(acc_ref, hsb_ref, h_ref, gf_ref):
    hv = jnp.maximum(acc_ref[0] + acc_ref[1] + hsb_ref[...], 0.0)
    h_ref[...] = hv

    @pl.when(pl.program_id(0) == 0)
    def _():
        gf_ref[...] = jnp.zeros_like(gf_ref)

    gf_ref[...] += jnp.sum(hv, axis=0, keepdims=True)


@jax.jit
def _tc_combine(accp, hsb):
    return pl.pallas_call(
        _combine_body,
        grid=(N // BN,),
        in_specs=[
            pl.BlockSpec((2, BN, D), lambda i: (0, i, 0)),
            pl.BlockSpec((BN, D), lambda i: (i, 0)),
        ],
        out_specs=[
            pl.BlockSpec((BN, D), lambda i: (i, 0)),
            pl.BlockSpec((1, D), lambda i: (0, 0)),
        ],
        out_shape=[
            jax.ShapeDtypeStruct((N, D), jnp.float32),
            jax.ShapeDtypeStruct((1, D), jnp.float32),
        ],
    )(accp, hsb)


NGROUP = C // 16


def _edge_body(qk_hbm, hw_hbm, src_hbm, dst_hbm, et_hbm, out_hbm,
               sbufs, dbufs, gbufs, dscats, qbufs, kbufs, mbufs, zbuf, acc,
               semi, semg, sems):
    cid = lax.axis_index("c")
    sid = lax.axis_index("s")
    wid = sid * NC + cid

    zv = jnp.zeros((16,), jnp.float32)

    def _zero_row(r, _):
        for j in range(D // 16):
            zbuf[r, pl.ds(j * 16, 16)] = zv
        return 0

    lax.fori_loop(0, ZROWS, _zero_row, 0)
    for p in range(ROWS_PER_TILE // ZROWS):
        pltpu.sync_copy(zbuf, acc.at[pl.ds(sid * ROWS_PER_TILE + p * ZROWS, ZROWS)])
    plsc.subcore_barrier()

    iota16 = lax.iota(jnp.int32, 16)

    def issue_idx(s, ch):
        base = wid * EPW + ch * C
        pltpu.async_copy(src_hbm.at[pl.ds(base, C)], sbufs[s], semi[s])
        pltpu.async_copy(dst_hbm.at[pl.ds(base, C)], dbufs[s], semi[s])
        pltpu.async_copy(et_hbm.at[pl.ds(base, C)], gbufs[s], semi[s])

    def wait_idx(s):
        pltpu.make_async_copy(src_hbm.at[pl.ds(0, C)], sbufs[s], semi[s]).wait()
        pltpu.make_async_copy(dst_hbm.at[pl.ds(0, C)], dbufs[s], semi[s]).wait()
        pltpu.make_async_copy(et_hbm.at[pl.ds(0, C)], gbufs[s], semi[s]).wait()

    def issue_gathers(s):
        for g in range(NGROUP):
            ev = gbufs[s][pl.ds(g * 16, 16)]
            sv = sbufs[s][pl.ds(g * 16, 16)]
            gbufs[s][pl.ds(g * 16, 16)] = ev * N + sv
        pltpu.async_copy(qk_hbm.at[sbufs[s]], qbufs[s], semg[s])
        pltpu.async_copy(qk_hbm.at[dbufs[s]], kbufs[s], semg[s])
        pltpu.async_copy(hw_hbm.at[gbufs[s]], mbufs[s], semg[s])

    def wait_gathers(s):
        pltpu.make_async_copy(qk_hbm.at[sbufs[s]], qbufs[s], semg[s]).wait()
        pltpu.make_async_copy(qk_hbm.at[dbufs[s]], kbufs[s], semg[s]).wait()
        pltpu.make_async_copy(hw_hbm.at[gbufs[s]], mbufs[s], semg[s]).wait()

    def compute(s):
        qbuf, kbuf, mbuf = qbufs[s], kbufs[s], mbufs[s]

        def _group(g, _):
            def _dot4(ii, zv):
                for t in range(4):
                    i = ii * 4 + t
                    e = g * 16 + i
                    p = jnp.zeros((16,), jnp.float32)
                    for j in range(DH // 16):
                        p = (p + qbuf[e, pl.ds(j * 16, 16)]
                             * kbuf[e, pl.ds(DH + j * 16, 16)])
                    zv = jnp.where(iota16 == i, jnp.sum(p), zv)
                return zv

            zv = lax.fori_loop(0, 4, _dot4, jnp.zeros((16,), jnp.float32))
            zv = zv * 0.125
            w = 1.0 / (1.0 + jnp.exp(-zv))
            for i in range(16):
                e = g * 16 + i
                wi = w[i]
                for j in range(D // 16):
                    mbuf[e, pl.ds(j * 16, 16)] = mbuf[e, pl.ds(j * 16, 16)] * wi
            return 0

        lax.fori_loop(0, NGROUP, _group, 0)

    def snap_dst(s):
        for g in range(NGROUP):
            dscats[s][pl.ds(g * 16, 16)] = dbufs[s][pl.ds(g * 16, 16)]

    def issue_scatter(s):
        pltpu.async_copy(mbufs[s], acc.at[dscats[s]], sems[s], add=True)

    def wait_scatter(s):
        pltpu.make_async_copy(mbufs[s], acc.at[dscats[s]], sems[s]).wait()

    NP_ = NCHUNK // 2
    issue_idx(0, 0)
    issue_idx(1, 1)
    wait_idx(0)
    issue_gathers(0)

    def _pair(p, _):
        wait_gathers(0)
        snap_dst(0)

        @pl.when(p > 0)
        def _():
            wait_scatter(1)
        wait_idx(1)
        issue_gathers(1)

        @pl.when(p < NP_ - 1)
        def _():
            issue_idx(0, 2 * p + 2)
        compute(0)
        issue_scatter(0)

        wait_gathers(1)
        snap_dst(1)
        wait_scatter(0)

        @pl.when(p < NP_ - 1)
        def _():
            wait_idx(0)
            issue_gathers(0)
            issue_idx(1, 2 * p + 3)
        compute(1)
        issue_scatter(1)
        return 0

    lax.fori_loop(0, NP_, _pair, 0)
    wait_scatter(1)

    plsc.subcore_barrier()
    pltpu.sync_copy(
        acc.at[pl.ds(sid * ROWS_PER_TILE, ROWS_PER_TILE)],
        out_hbm.at[cid, pl.ds(sid * ROWS_PER_TILE, ROWS_PER_TILE)],
    )


@jax.jit
def _sc_edge(qk, hw_flat, src, dst, et):
    mesh = plsc.VectorSubcoreMesh(core_axis_name="c", subcore_axis_name="s")
    idx_t = pltpu.VMEM((C,), jnp.int32)
    row_t = pltpu.VMEM((C, D), jnp.float32)
    f = functools.partial(
        pl.kernel,
        mesh=mesh,
        compiler_params=pltpu.CompilerParams(needs_layout_passes=False),
        out_type=jax.ShapeDtypeStruct((2, NPAD, D), jnp.float32),
        scratch_types=[
            (idx_t, idx_t),
            (idx_t, idx_t),
            (idx_t, idx_t),
            (idx_t, idx_t),
            (row_t, row_t),
            (row_t, row_t),
            (row_t, row_t),
            pltpu.VMEM((ZROWS, D), jnp.float32),
            pltpu.VMEM_SHARED((NPAD, D), jnp.float32),
            (pltpu.SemaphoreType.DMA, pltpu.SemaphoreType.DMA),
            (pltpu.SemaphoreType.DMA, pltpu.SemaphoreType.DMA),
            (pltpu.SemaphoreType.DMA, pltpu.SemaphoreType.DMA),
        ],
    )(_edge_body)
    return f(qk, hw_flat, src, dst, et)


def kernel(x, Wq0, Wk0, Wrel0, b0, Ws0, Wq1, Wk1, Wrel1, b1, Ws1,
           Wq2, Wk2, Wrel2, b2, Ws2, edge_index, edge_type):
    npad_e = EP - E
    src = jnp.concatenate(
        [edge_index[0].astype(jnp.int32), jnp.zeros((npad_e,), jnp.int32)])
    dst = jnp.concatenate(
        [edge_index[1].astype(jnp.int32),
         jnp.full((npad_e,), NPAD - 1, jnp.int32)])
    et = jnp.concatenate(
        [edge_type.astype(jnp.int32), jnp.zeros((npad_e,), jnp.int32)])

    wqs = jnp.stack([Wq0, Wq1, Wq2])
    wks = jnp.stack([Wk0, Wk1, Wk2])
    wrels = jnp.stack([Wrel0.reshape(R, D, D), Wrel1.reshape(R, D, D),
                       Wrel2.reshape(R, D, D)])
    bs = jnp.stack([b0.reshape(1, D), b1.reshape(1, D), b2.reshape(1, D)])
    wss = jnp.stack([Ws0, Ws1, Ws2])

    def _layer_step(carry, ws):
        h, _ = carry
        wq, wk, wrel, b2, w_s = ws
        qk, hw, hsb = _tc_pre(h, wq, wk, wrel, w_s, b2)
        accp = _sc_edge(qk, hw.reshape(R * N, D), src, dst, et)
        h_new, gf = _tc_combine(accp, hsb)
        return (h_new, gf), None

    gf0 = jnp.zeros((1, D), jnp.float32)
    (h, gf), _ = lax.scan(_layer_step, (x, gf0),
                          (wqs, wks, wrels, bs, wss))
    return gf, h
